# baseline (speedup 1.0000x reference)
import functools

import jax
import jax.numpy as jnp
from jax import lax
from jax.experimental import pallas as pl
from jax.experimental.pallas import tpu as pltpu


def _tables_body(nodes_ref, wes_ref, wer_ref, be_ref, wa_ref,
                 ps_ref, pr_ref, a_ref):
    nd = nodes_ref[...]
    ps = jnp.dot(nd, wes_ref[...].T, preferred_element_type=jnp.float32)
    pr = jnp.dot(nd, wer_ref[...].T, preferred_element_type=jnp.float32)
    pr = pr + be_ref[...][None, :]
    a = jnp.dot(nd, wa_ref[...].T, preferred_element_type=jnp.float32)
    a = a - jnp.max(a, axis=0, keepdims=True)
    ps_ref[0] = ps[:, :64]
    ps_ref[1] = ps[:, 64:]
    pr_ref[0] = pr[:, :64]
    pr_ref[1] = pr[:, 64:]
    a_ref[...] = a


def _node_tables(nodes, W_edge, b_edge, W_attn):
    n = nodes.shape[0]
    wes = W_edge[:, 16:144]
    wer = W_edge[:, 144:272]
    wa = W_attn[0].reshape(2, 128)
    return pl.pallas_call(
        _tables_body,
        out_shape=(
            jax.ShapeDtypeStruct((2, n, 64), jnp.float32),
            jax.ShapeDtypeStruct((2, n, 64), jnp.float32),
            jax.ShapeDtypeStruct((n, 2), jnp.float32),
        ),
    )(nodes, wes, wer, b_edge, wa)


def _q_body(e_ref, wq_ref, q_ref):
    q = jnp.dot(e_ref[...], wq_ref[...], preferred_element_type=jnp.float32)
    q_ref[0] = q[:, :64]
    q_ref[1] = q[:, 64:]


def _edge_q(edges, W_edge, blk=8000):
    e = edges.shape[0]
    wq = W_edge[:, :16].T
    return pl.pallas_call(
        _q_body,
        grid=(e // blk,),
        in_specs=[
            pl.BlockSpec((blk, 16), lambda i: (i, 0)),
            pl.BlockSpec((16, 128), lambda i: (0, 0)),
        ],
        out_specs=pl.BlockSpec((2, blk, 64), lambda i: (0, i, 0)),
        out_shape=jax.ShapeDtypeStruct((2, e, 64), jnp.float32),
    )(edges, wq)


def _nodeupd_body(nodes_ref, agg_ref, wn_ref, bn_ref, o_ref):
    nd = nodes_ref[...]
    wn = wn_ref[...]
    acc = jnp.dot(nd, wn[:, :128].T, preferred_element_type=jnp.float32)
    acc += jnp.dot(agg_ref[0, 0], wn[:, 128:192].T,
                   preferred_element_type=jnp.float32)
    acc += jnp.dot(agg_ref[1, 0], wn[:, 192:256].T,
                   preferred_element_type=jnp.float32)
    acc += jnp.dot(agg_ref[0, 1], wn[:, 256:320].T,
                   preferred_element_type=jnp.float32)
    acc += jnp.dot(agg_ref[1, 1], wn[:, 320:384].T,
                   preferred_element_type=jnp.float32)
    o_ref[...] = jax.nn.relu(acc + bn_ref[...][None, :])


def _node_update(nodes, agg, W_node, b_node):
    n = nodes.shape[0]
    return pl.pallas_call(
        _nodeupd_body,
        out_shape=jax.ShapeDtypeStruct((n, 128), jnp.float32),
    )(nodes, agg, W_node, b_node)


def _edge_middle(senders, receivers, a_tbl, ps, pr, q):
    n = a_tbl.shape[0]
    logits = a_tbl[senders, 0] + a_tbl[receivers, 1]
    p = jnp.exp(logits)
    denom = jax.ops.segment_sum(p, receivers, num_segments=n)
    scale = (p / denom[receivers])[:, None]
    w0 = jax.nn.relu(q[0] + ps[0][senders] + pr[0][receivers]) * scale
    w1 = jax.nn.relu(q[1] + ps[1][senders] + pr[1][receivers]) * scale
    w_edges = jnp.concatenate([w0, w1], axis=-1)
    agg = jnp.stack([
        jnp.stack([jax.ops.segment_sum(w0, senders, num_segments=n),
                   jax.ops.segment_sum(w0, receivers, num_segments=n)]),
        jnp.stack([jax.ops.segment_sum(w1, senders, num_segments=n),
                   jax.ops.segment_sum(w1, receivers, num_segments=n)]),
    ])
    return w_edges, agg


def kernel(nodes, edges, senders, receivers,
           W_edge, b_edge, W_node, b_node, W_attn, b_attn):
    ps, pr, a_tbl = _node_tables(nodes, W_edge, b_edge, W_attn)
    q = _edge_q(edges, W_edge)
    w_edges, agg = _edge_middle(senders, receivers, a_tbl, ps, pr, q)
    upd_n = _node_update(nodes, agg, W_node, b_node)
    return (upd_n, w_edges)


# speedup vs baseline: 4.6746x; 4.6746x over previous
import dataclasses
import functools

import jax
import jax.numpy as jnp
from jax import lax
from jax.experimental import pallas as pl
from jax.experimental.pallas import tpu as pltpu
from jax.experimental.pallas import tpu_sc as plsc


def _tables_body(nodes_ref, wes_ref, wer_ref, be_ref, wa_ref,
                 ps_ref, pr_ref, as_ref, ar_ref):
    nd = nodes_ref[...]
    ps = jnp.dot(nd, wes_ref[...].T, preferred_element_type=jnp.float32)
    pr = jnp.dot(nd, wer_ref[...].T, preferred_element_type=jnp.float32)
    pr = pr + be_ref[...][None, :]
    a = jnp.dot(nd, wa_ref[...].T, preferred_element_type=jnp.float32)
    a = a - jnp.max(a, axis=0, keepdims=True)
    ps_ref[0] = ps[:, :64]
    ps_ref[1] = ps[:, 64:]
    pr_ref[0] = pr[:, :64]
    pr_ref[1] = pr[:, 64:]
    as_ref[...] = jnp.broadcast_to(a[:, 0:1], a_ref_shape := (a.shape[0], 16))
    ar_ref[...] = jnp.broadcast_to(a[:, 1:2], a_ref_shape)


def _node_tables(nodes, W_edge, b_edge, W_attn):
    n = nodes.shape[0]
    wes = W_edge[:, 16:144]
    wer = W_edge[:, 144:272]
    wa = W_attn[0].reshape(2, 128)
    return pl.pallas_call(
        _tables_body,
        out_shape=(
            jax.ShapeDtypeStruct((2, n, 64), jnp.float32),
            jax.ShapeDtypeStruct((2, n, 64), jnp.float32),
            jax.ShapeDtypeStruct((n, 16), jnp.float32),
            jax.ShapeDtypeStruct((n, 16), jnp.float32),
        ),
    )(nodes, wes, wer, b_edge, wa)


def _q_body(e_ref, wq_ref, q_ref):
    q = jnp.dot(e_ref[...], wq_ref[...], preferred_element_type=jnp.float32)
    q_ref[0] = q[:, :64]
    q_ref[1] = q[:, 64:]


def _edge_q(edges, W_edge, blk=8000):
    e = edges.shape[0]
    wq = W_edge[:, :16].T
    return pl.pallas_call(
        _q_body,
        grid=(e // blk,),
        in_specs=[
            pl.BlockSpec((blk, 16), lambda i: (i, 0)),
            pl.BlockSpec((16, 128), lambda i: (0, 0)),
        ],
        out_specs=pl.BlockSpec((2, blk, 64), lambda i: (0, i, 0)),
        out_shape=jax.ShapeDtypeStruct((2, e, 64), jnp.float32),
    )(edges, wq)


def _nodeupd_body(nodes_ref, agg_ref, wn_ref, bn_ref, o_ref):
    nd = nodes_ref[...]
    n = nd.shape[0]
    wn = wn_ref[...]
    acc = jnp.dot(nd, wn[:, :128].T, preferred_element_type=jnp.float32)
    acc += jnp.dot(agg_ref[0, 0][:n], wn[:, 128:192].T,
                   preferred_element_type=jnp.float32)
    acc += jnp.dot(agg_ref[1, 0][:n], wn[:, 192:256].T,
                   preferred_element_type=jnp.float32)
    acc += jnp.dot(agg_ref[0, 1][:n], wn[:, 256:320].T,
                   preferred_element_type=jnp.float32)
    acc += jnp.dot(agg_ref[1, 1][:n], wn[:, 320:384].T,
                   preferred_element_type=jnp.float32)
    o_ref[...] = jax.nn.relu(acc + bn_ref[...][None, :])


def _node_update(nodes, agg, W_node, b_node):
    n = nodes.shape[0]
    return pl.pallas_call(
        _nodeupd_body,
        out_shape=jax.ShapeDtypeStruct((n, 128), jnp.float32),
    )(nodes, agg, W_node, b_node)


_NT = 16
_L = 16
_B = 40


def _edge_middle(senders, receivers, as16, ar16, ps, pr, q):
    n = as16.shape[0]
    e = senders.shape[0]
    np_ = ((n + 639) // 640) * 640
    st = np_ // _NT
    et = e // _NT
    nblk = et // _B
    ps0, ps1 = ps[0], ps[1]
    pr0, pr1 = pr[0], pr[1]
    mesh = plsc.VectorSubcoreMesh(core_axis_name="c", subcore_axis_name="s")
    cp = pltpu.CompilerParams(use_tc_tiling_on_sc=False)
    if "needs_layout_passes" in pltpu.CompilerParams.__dataclass_fields__:
        cp = dataclasses.replace(cp, needs_layout_passes=False)

    @functools.partial(
        pl.kernel,
        compiler_params=cp,
        out_type=[
            jax.ShapeDtypeStruct((2, e, 64), jnp.float32),
            jax.ShapeDtypeStruct((2, 2, np_, 64), jnp.float32),
        ],
        mesh=mesh,
        scratch_types=[
            pltpu.VMEM((_B,), jnp.int32),
            pltpu.VMEM((_B,), jnp.int32),
            pltpu.VMEM((_B, _L), jnp.float32),
            pltpu.VMEM((_B, _L), jnp.float32),
            pltpu.VMEM((_B, _L), jnp.float32),
            pltpu.VMEM((_B, _L), jnp.float32),
            pltpu.VMEM((_B, 64), jnp.float32),
            pltpu.VMEM((_B, 64), jnp.float32),
            pltpu.VMEM((_B, 64), jnp.float32),
            pltpu.VMEM((_B, 64), jnp.float32),
            pltpu.VMEM_SHARED((np_, _L), jnp.float32),
            pltpu.VMEM_SHARED((np_, 64), jnp.float32),
            pltpu.VMEM_SHARED((np_, 64), jnp.float32),
            pltpu.SemaphoreType.DMA,
            pltpu.SemaphoreType.DMA,
            pltpu.SemaphoreType.DMA,
            pltpu.SemaphoreType.DMA,
            pltpu.SemaphoreType.DMA,
        ],
    )
    def sc_kernel(s_hbm, r_hbm, as_hbm, ar_hbm, ps0_hbm, ps1_hbm,
                  pr0_hbm, pr1_hbm, q_hbm,
                  w_hbm, agg_hbm,
                  s_v, r_v, asg_v, arg_v, pw_v, dn_v,
                  q_v, gs_v, gr_v, w_v,
                  denom_sh, sent_sh, recv_sh, sem1, sem2, sem3, sem4, sem5):
        c = lax.axis_index("c")
        s = lax.axis_index("s")
        t0 = s * et
        zero16 = jnp.zeros((_L,), jnp.float32)

        @pl.loop(0, _B)
        def _(i):
            for j in range(4):
                w_v[i, pl.ds(j * _L, _L)] = zero16
            pw_v[i, pl.ds(0, _L)] = zero16

        r0 = s * st
        for k in range(st // _B):
            pltpu.sync_copy(pw_v, denom_sh.at[pl.ds(r0 + k * _B, _B)])
            pltpu.sync_copy(w_v, sent_sh.at[pl.ds(r0 + k * _B, _B)])
            pltpu.sync_copy(w_v, recv_sh.at[pl.ds(r0 + k * _B, _B)])

        plsc.subcore_barrier()

        @pl.loop(0, nblk)
        def _(i):
            base = t0 + i * _B
            pltpu.sync_copy(s_hbm.at[pl.ds(base, _B)], s_v)
            pltpu.sync_copy(r_hbm.at[pl.ds(base, _B)], r_v)
            g1 = pltpu.async_copy(as_hbm.at[s_v], asg_v, sem1)
            g2 = pltpu.async_copy(ar_hbm.at[r_v], arg_v, sem2)
            g1.wait()
            g2.wait()

            @pl.loop(0, _B)
            def _(ed):
                pw_v[ed, pl.ds(0, _L)] = jnp.exp(
                    asg_v[ed, pl.ds(0, _L)] + arg_v[ed, pl.ds(0, _L)])

            pltpu.sync_copy(pw_v, denom_sh.at[r_v], add=True)

        plsc.subcore_barrier()

        @pl.loop(0, nblk)
        def _(i):
            base = t0 + i * _B
            pltpu.sync_copy(s_hbm.at[pl.ds(base, _B)], s_v)
            pltpu.sync_copy(r_hbm.at[pl.ds(base, _B)], r_v)

            @pl.when(c == 0)
            def _():
                pltpu.async_copy(ps0_hbm.at[s_v], gs_v, sem1).wait()
                pltpu.async_copy(pr0_hbm.at[r_v], gr_v, sem2).wait()

            @pl.when(c == 1)
            def _():
                pltpu.async_copy(ps1_hbm.at[s_v], gs_v, sem1).wait()
                pltpu.async_copy(pr1_hbm.at[r_v], gr_v, sem2).wait()

            g1 = pltpu.async_copy(as_hbm.at[s_v], asg_v, sem3)
            g2 = pltpu.async_copy(ar_hbm.at[r_v], arg_v, sem4)
            g3 = pltpu.async_copy(denom_sh.at[r_v], dn_v, sem5)
            pltpu.sync_copy(q_hbm.at[c, pl.ds(base, _B)], q_v)
            g1.wait()
            g2.wait()
            g3.wait()

            @pl.loop(0, _B)
            def _(ed):
                l16 = pl.ds(0, _L)
                sv = (jnp.exp(asg_v[ed, l16] + arg_v[ed, l16])
                      / dn_v[ed, l16])
                for j in range(4):
                    sl = pl.ds(j * _L, _L)
                    x = q_v[ed, sl] + gs_v[ed, sl] + gr_v[ed, sl]
                    w_v[ed, sl] = jnp.maximum(x, 0.0) * sv

            pltpu.sync_copy(w_v, w_hbm.at[c, pl.ds(base, _B)])
            pltpu.sync_copy(w_v, sent_sh.at[s_v], add=True)
            pltpu.sync_copy(w_v, recv_sh.at[r_v], add=True)

        plsc.subcore_barrier()

        pltpu.sync_copy(sent_sh.at[pl.ds(r0, st)],
                        agg_hbm.at[c, 0, pl.ds(r0, st)])
        pltpu.sync_copy(recv_sh.at[pl.ds(r0, st)],
                        agg_hbm.at[c, 1, pl.ds(r0, st)])

    return sc_kernel(senders, receivers, as16, ar16, ps0, ps1, pr0, pr1, q)


def _wcat_body(w2_ref, o_ref):
    o_ref[:, :64] = w2_ref[0]
    o_ref[:, 64:] = w2_ref[1]


def _w_concat(w2, blk=8000):
    e = w2.shape[1]
    return pl.pallas_call(
        _wcat_body,
        grid=(e // blk,),
        in_specs=[pl.BlockSpec((2, blk, 64), lambda i: (0, i, 0))],
        out_specs=pl.BlockSpec((blk, 128), lambda i: (i, 0)),
        out_shape=jax.ShapeDtypeStruct((e, 128), jnp.float32),
    )(w2)


def kernel(nodes, edges, senders, receivers,
           W_edge, b_edge, W_node, b_node, W_attn, b_attn):
    ps, pr, as16, ar16 = _node_tables(nodes, W_edge, b_edge, W_attn)
    q = _edge_q(edges, W_edge)
    w2, agg = _edge_middle(senders, receivers, as16, ar16, ps, pr, q)
    w_edges = _w_concat(w2)
    upd_n = _node_update(nodes, agg, W_node, b_node)
    return (upd_n, w_edges)


# speedup vs baseline: 6.3604x; 1.3607x over previous
import dataclasses
import functools

import jax
import jax.numpy as jnp
from jax import lax
from jax.experimental import pallas as pl
from jax.experimental.pallas import tpu as pltpu
from jax.experimental.pallas import tpu_sc as plsc


def _tables_body(nodes_ref, wes_ref, wer_ref, be_ref, wa_ref,
                 ps_ref, pr_ref, as_ref, ar_ref):
    nd = nodes_ref[...]
    ps = jnp.dot(nd, wes_ref[...].T, preferred_element_type=jnp.float32)
    pr = jnp.dot(nd, wer_ref[...].T, preferred_element_type=jnp.float32)
    pr = pr + be_ref[...][None, :]
    a = jnp.dot(nd, wa_ref[...].T, preferred_element_type=jnp.float32)
    a = a - jnp.max(a, axis=0, keepdims=True)
    ps_ref[0] = ps[:, :64]
    ps_ref[1] = ps[:, 64:]
    pr_ref[0] = pr[:, :64]
    pr_ref[1] = pr[:, 64:]
    as_ref[...] = jnp.broadcast_to(a[:, 0:1], a_ref_shape := (a.shape[0], 16))
    ar_ref[...] = jnp.broadcast_to(a[:, 1:2], a_ref_shape)


def _node_tables(nodes, W_edge, b_edge, W_attn):
    n = nodes.shape[0]
    wes = W_edge[:, 16:144]
    wer = W_edge[:, 144:272]
    wa = W_attn[0].reshape(2, 128)
    return pl.pallas_call(
        _tables_body,
        out_shape=(
            jax.ShapeDtypeStruct((2, n, 64), jnp.float32),
            jax.ShapeDtypeStruct((2, n, 64), jnp.float32),
            jax.ShapeDtypeStruct((n, 16), jnp.float32),
            jax.ShapeDtypeStruct((n, 16), jnp.float32),
        ),
    )(nodes, wes, wer, b_edge, wa)


def _q_body(e_ref, wq_ref, q_ref):
    q = jnp.dot(e_ref[...], wq_ref[...], preferred_element_type=jnp.float32)
    q_ref[0] = q[:, :64]
    q_ref[1] = q[:, 64:]


def _edge_q(edges, W_edge, blk=8000):
    e = edges.shape[0]
    wq = W_edge[:, :16].T
    return pl.pallas_call(
        _q_body,
        grid=(e // blk,),
        in_specs=[
            pl.BlockSpec((blk, 16), lambda i: (i, 0)),
            pl.BlockSpec((16, 128), lambda i: (0, 0)),
        ],
        out_specs=pl.BlockSpec((2, blk, 64), lambda i: (0, i, 0)),
        out_shape=jax.ShapeDtypeStruct((2, e, 64), jnp.float32),
    )(edges, wq)


def _nodeupd_body(nodes_ref, agg_ref, wn_ref, bn_ref, o_ref):
    nd = nodes_ref[...]
    n = nd.shape[0]
    wn = wn_ref[...]
    acc = jnp.dot(nd, wn[:, :128].T, preferred_element_type=jnp.float32)
    acc += jnp.dot(agg_ref[0, 0][:n], wn[:, 128:192].T,
                   preferred_element_type=jnp.float32)
    acc += jnp.dot(agg_ref[1, 0][:n], wn[:, 192:256].T,
                   preferred_element_type=jnp.float32)
    acc += jnp.dot(agg_ref[0, 1][:n], wn[:, 256:320].T,
                   preferred_element_type=jnp.float32)
    acc += jnp.dot(agg_ref[1, 1][:n], wn[:, 320:384].T,
                   preferred_element_type=jnp.float32)
    o_ref[...] = jax.nn.relu(acc + bn_ref[...][None, :])


def _node_update(nodes, agg, W_node, b_node):
    n = nodes.shape[0]
    return pl.pallas_call(
        _nodeupd_body,
        out_shape=jax.ShapeDtypeStruct((n, 128), jnp.float32),
    )(nodes, agg, W_node, b_node)


_NT = 16
_L = 16
_B = 80


def _edge_middle(senders, receivers, as16, ar16, ps, pr, q):
    n = as16.shape[0]
    e = senders.shape[0]
    np_ = ((n + 639) // 640) * 640
    st = np_ // _NT
    et = e // _NT
    nblk = et // _B
    ps0, ps1 = ps[0], ps[1]
    pr0, pr1 = pr[0], pr[1]
    mesh = plsc.VectorSubcoreMesh(core_axis_name="c", subcore_axis_name="s")
    cp = pltpu.CompilerParams(use_tc_tiling_on_sc=False)
    if "needs_layout_passes" in pltpu.CompilerParams.__dataclass_fields__:
        cp = dataclasses.replace(cp, needs_layout_passes=False)

    @functools.partial(
        pl.kernel,
        compiler_params=cp,
        out_type=[
            jax.ShapeDtypeStruct((2, e, 64), jnp.float32),
            jax.ShapeDtypeStruct((2, 2, np_, 64), jnp.float32),
        ],
        mesh=mesh,
        scratch_types=[
            pltpu.VMEM((_B,), jnp.int32),
            pltpu.VMEM((_B,), jnp.int32),
            pltpu.VMEM((_B, _L), jnp.float32),
            pltpu.VMEM((_B, _L), jnp.float32),
            pltpu.VMEM((_B, _L), jnp.float32),
            pltpu.VMEM((_B, _L), jnp.float32),
            pltpu.VMEM((_B, 64), jnp.float32),
            pltpu.VMEM((_B, 64), jnp.float32),
            pltpu.VMEM((_B, 64), jnp.float32),
            pltpu.VMEM((_B, 64), jnp.float32),
            pltpu.VMEM_SHARED((np_, _L), jnp.float32),
            pltpu.VMEM_SHARED((np_, 64), jnp.float32),
            pltpu.VMEM_SHARED((np_, 64), jnp.float32),
            pltpu.SemaphoreType.DMA,
            pltpu.SemaphoreType.DMA,
            pltpu.SemaphoreType.DMA,
            pltpu.SemaphoreType.DMA,
            pltpu.SemaphoreType.DMA,
        ],
    )
    def sc_kernel(s_hbm, r_hbm, as_hbm, ar_hbm, ps0_hbm, ps1_hbm,
                  pr0_hbm, pr1_hbm, q_hbm,
                  w_hbm, agg_hbm,
                  s_v, r_v, asg_v, arg_v, pw_v, dn_v,
                  q_v, gs_v, gr_v, w_v,
                  denom_sh, sent_sh, recv_sh, sem1, sem2, sem3, sem4, sem5):
        c = lax.axis_index("c")
        s = lax.axis_index("s")
        t0 = s * et
        zero16 = jnp.zeros((_L,), jnp.float32)

        @pl.loop(0, _B)
        def _(i):
            for j in range(4):
                w_v[i, pl.ds(j * _L, _L)] = zero16
            pw_v[i, pl.ds(0, _L)] = zero16

        r0 = s * st
        for k in range(st // _B):
            pltpu.sync_copy(pw_v, denom_sh.at[pl.ds(r0 + k * _B, _B)])
            pltpu.sync_copy(w_v, sent_sh.at[pl.ds(r0 + k * _B, _B)])
            pltpu.sync_copy(w_v, recv_sh.at[pl.ds(r0 + k * _B, _B)])

        plsc.subcore_barrier()

        @pl.loop(0, nblk)
        def _(i):
            base = t0 + i * _B
            pltpu.sync_copy(s_hbm.at[pl.ds(base, _B)], s_v)
            pltpu.sync_copy(r_hbm.at[pl.ds(base, _B)], r_v)
            g1 = pltpu.async_copy(as_hbm.at[s_v], asg_v, sem1)
            g2 = pltpu.async_copy(ar_hbm.at[r_v], arg_v, sem2)
            g1.wait()
            g2.wait()

            @pl.loop(0, _B)
            def _(ed):
                pw_v[ed, pl.ds(0, _L)] = jnp.exp(
                    asg_v[ed, pl.ds(0, _L)] + arg_v[ed, pl.ds(0, _L)])

            pltpu.sync_copy(pw_v, denom_sh.at[r_v], add=True)

        plsc.subcore_barrier()

        @pl.loop(0, nblk)
        def _(i):
            base = t0 + i * _B
            pltpu.sync_copy(s_hbm.at[pl.ds(base, _B)], s_v)
            pltpu.sync_copy(r_hbm.at[pl.ds(base, _B)], r_v)

            @pl.when(c == 0)
            def _():
                pltpu.async_copy(ps0_hbm.at[s_v], gs_v, sem1).wait()
                pltpu.async_copy(pr0_hbm.at[r_v], gr_v, sem2).wait()

            @pl.when(c == 1)
            def _():
                pltpu.async_copy(ps1_hbm.at[s_v], gs_v, sem1).wait()
                pltpu.async_copy(pr1_hbm.at[r_v], gr_v, sem2).wait()

            g1 = pltpu.async_copy(as_hbm.at[s_v], asg_v, sem3)
            g2 = pltpu.async_copy(ar_hbm.at[r_v], arg_v, sem4)
            g3 = pltpu.async_copy(denom_sh.at[r_v], dn_v, sem5)
            pltpu.sync_copy(q_hbm.at[c, pl.ds(base, _B)], q_v)
            g1.wait()
            g2.wait()
            g3.wait()

            @pl.loop(0, _B)
            def _(ed):
                l16 = pl.ds(0, _L)
                sv = (jnp.exp(asg_v[ed, l16] + arg_v[ed, l16])
                      / dn_v[ed, l16])
                for j in range(4):
                    sl = pl.ds(j * _L, _L)
                    x = q_v[ed, sl] + gs_v[ed, sl] + gr_v[ed, sl]
                    w_v[ed, sl] = jnp.maximum(x, 0.0) * sv

            pltpu.sync_copy(w_v, w_hbm.at[c, pl.ds(base, _B)])
            pltpu.sync_copy(w_v, sent_sh.at[s_v], add=True)
            pltpu.sync_copy(w_v, recv_sh.at[r_v], add=True)

        plsc.subcore_barrier()

        pltpu.sync_copy(sent_sh.at[pl.ds(r0, st)],
                        agg_hbm.at[c, 0, pl.ds(r0, st)])
        pltpu.sync_copy(recv_sh.at[pl.ds(r0, st)],
                        agg_hbm.at[c, 1, pl.ds(r0, st)])

    return sc_kernel(senders, receivers, as16, ar16, ps0, ps1, pr0, pr1, q)


def _wcat_body(w2_ref, o_ref):
    o_ref[:, :64] = w2_ref[0]
    o_ref[:, 64:] = w2_ref[1]


def _w_concat(w2, blk=8000):
    e = w2.shape[1]
    return pl.pallas_call(
        _wcat_body,
        grid=(e // blk,),
        in_specs=[pl.BlockSpec((2, blk, 64), lambda i: (0, i, 0))],
        out_specs=pl.BlockSpec((blk, 128), lambda i: (i, 0)),
        out_shape=jax.ShapeDtypeStruct((e, 128), jnp.float32),
    )(w2)


def kernel(nodes, edges, senders, receivers,
           W_edge, b_edge, W_node, b_node, W_attn, b_attn):
    ps, pr, as16, ar16 = _node_tables(nodes, W_edge, b_edge, W_attn)
    q = _edge_q(edges, W_edge)
    w2, agg = _edge_middle(senders, receivers, as16, ar16, ps, pr, q)
    w_edges = _w_concat(w2)
    upd_n = _node_update(nodes, agg, W_node, b_node)
    return (upd_n, w_edges)
